# baseline (speedup 1.0000x reference)
import functools

import jax
import jax.numpy as jnp
from jax import lax
from jax.experimental import pallas as pl
from jax.experimental.pallas import tpu as pltpu
from jax.experimental.pallas import tpu_sc as plsc

B = 16384
EMB = 16
N_DENSE = 13

NC, NS = 2, 16
NW = NC * NS
BPW = B // NW
CHUNK = 128
NCHUNK = BPW // CHUNK


def _sc_gather_body(e_base, e_level, e_sex, e_tag, idx_all, out,
                    idx_v, rows_v, sem):
    wid = lax.axis_index("s") * NC + lax.axis_index("c")
    base = wid * BPW
    pltpu.sync_copy(idx_all.at[:, wid], idx_v)
    tables = (e_base, e_level, e_sex, e_tag)
    copies = []
    for f in range(4):
        for j in range(NCHUNK):
            copies.append(pltpu.async_copy(
                tables[f].at[idx_v.at[f, j]],
                rows_v.at[f, pl.ds(j * CHUNK, CHUNK)],
                sem))
    for c in copies:
        c.wait()
    for f in range(4):
        pltpu.sync_copy(rows_v.at[f], out.at[f, pl.ds(base, BPW)])


def _sc_gather(e_base, e_level, e_sex, e_tag, idx_all):
    mesh = plsc.VectorSubcoreMesh(core_axis_name="c", subcore_axis_name="s")
    return pl.kernel(
        _sc_gather_body,
        mesh=mesh,
        out_type=jax.ShapeDtypeStruct((4, B, EMB), jnp.float32),
        scratch_types=[
            pltpu.VMEM((4, NCHUNK, CHUNK), jnp.int32),
            pltpu.VMEM((4, BPW, EMB), jnp.float32),
            pltpu.SemaphoreType.DMA,
        ],
        compiler_params=pltpu.CompilerParams(use_tc_tiling_on_sc=False),
    )(e_base, e_level, e_sex, e_tag, idx_all)


BLK = 2048
GRID = B // BLK


def _tc_dense_body(emb_ref, dense_ref, target_ref,
                   w1_ref, b1_ref, w2_ref, b2_ref, w3_ref,
                   pred_ref, loss_ref):
    i = pl.program_id(0)
    e0 = emb_ref[0]
    e1 = emb_ref[1]
    e2 = emb_ref[2]
    e3 = emb_ref[3]
    s = e0 + e1 + e2 + e3
    sq = e0 * e0 + e1 * e1 + e2 * e2 + e3 * e3
    cross = 0.5 * jnp.sum(s * s - sq, axis=1)
    w1 = w1_ref[...]
    h = (jnp.dot(e0, w1[0:16], preferred_element_type=jnp.float32)
         + jnp.dot(e1, w1[16:32], preferred_element_type=jnp.float32)
         + jnp.dot(e2, w1[32:48], preferred_element_type=jnp.float32)
         + jnp.dot(e3, w1[48:64], preferred_element_type=jnp.float32)
         + jnp.dot(dense_ref[...], w1[64:77],
                   preferred_element_type=jnp.float32)
         + b1_ref[...][None, :])
    h = jnp.maximum(h, 0.0)
    h = jnp.dot(h, w2_ref[...], preferred_element_type=jnp.float32) \
        + b2_ref[...][None, :]
    h = jnp.maximum(h, 0.0)
    logit = jnp.dot(h, w3_ref[...],
                    preferred_element_type=jnp.float32)[:, 0] + cross
    pred = 1.0 / (1.0 + jnp.exp(-logit))
    pred_ref[...] = pred
    p = jnp.clip(pred, 1e-7, 1.0 - 1e-7)
    t = target_ref[...]
    partial = -jnp.sum(
        t * jnp.log(p) + (1.0 - t) * jnp.log(1.0 - p)).reshape(1, 1)

    @pl.when(i == 0)
    def _():
        loss_ref[...] = jnp.zeros((1, 1), jnp.float32)

    loss_ref[...] += partial

    @pl.when(i == GRID - 1)
    def _():
        loss_ref[...] = loss_ref[...] * (1.0 / B)


def _tc_dense(emb, dense_features, target, w1, b1, w2, b2, w3):
    return pl.pallas_call(
        _tc_dense_body,
        grid=(GRID,),
        in_specs=[
            pl.BlockSpec((4, BLK, EMB), lambda i: (0, i, 0)),
            pl.BlockSpec((BLK, N_DENSE), lambda i: (i, 0)),
            pl.BlockSpec((BLK,), lambda i: (i,)),
            pl.BlockSpec((77, 5), lambda i: (0, 0)),
            pl.BlockSpec((5,), lambda i: (0,)),
            pl.BlockSpec((5, 2), lambda i: (0, 0)),
            pl.BlockSpec((2,), lambda i: (0,)),
            pl.BlockSpec((2, 1), lambda i: (0, 0)),
        ],
        out_specs=[
            pl.BlockSpec((BLK,), lambda i: (i,)),
            pl.BlockSpec((1, 1), lambda i: (0, 0)),
        ],
        out_shape=[
            jax.ShapeDtypeStruct((B,), jnp.float32),
            jax.ShapeDtypeStruct((1, 1), jnp.float32),
        ],
    )(emb, dense_features, target, w1, b1, w2, b2, w3)


def kernel(base_cd, level, sex, tag, dense_features, target,
           E_base, E_level, E_sex, E_tag, W1, b1, W2, b2, W3):
    idx_all = jnp.stack([
        base_cd.astype(jnp.int32),
        level.astype(jnp.int32),
        sex.astype(jnp.int32),
        tag.astype(jnp.int32),
    ]).reshape(4, NW, NCHUNK, CHUNK)
    emb = _sc_gather(E_base, E_level, E_sex, E_tag, idx_all)
    pred, loss = _tc_dense(emb, dense_features, target, W1, b1, W2, b2, W3)
    return (pred, loss[0, 0])


# speedup vs baseline: 1.0827x; 1.0827x over previous
import functools

import jax
import jax.numpy as jnp
from jax import lax
from jax.experimental import pallas as pl
from jax.experimental.pallas import tpu as pltpu
from jax.experimental.pallas import tpu_sc as plsc

B = 16384
EMB = 16
N_DENSE = 13

NC, NS = 2, 16
NW = NC * NS
BPW = B // NW
GROUPS = BPW // 16
PR = B // 8
PRW = BPW // 8


def _sc_gather_body(e_base, e_level, e_sex, e_tag,
                    i_base, i_level, i_sex, i_tag, dn16,
                    emb_out, dnp_out,
                    idx_v, rows_v, dnp_v, sem):
    wid = lax.axis_index("s") * NC + lax.axis_index("c")
    base = wid * BPW
    orow = wid * PRW
    pltpu.sync_copy(i_base.at[pl.ds(base, BPW)], idx_v.at[0])
    pltpu.sync_copy(i_level.at[pl.ds(base, BPW)], idx_v.at[1])
    pltpu.sync_copy(i_sex.at[pl.ds(base, BPW)], idx_v.at[2])
    pltpu.sync_copy(i_tag.at[pl.ds(base, BPW)], idx_v.at[3])
    tables = (e_base, e_level, e_sex, e_tag)

    def step(g, carry):
        copies = []
        for f in range(4):
            vec = idx_v[f, pl.ds(g * 16, 16)]
            for l in range(16):
                row = g * 2 + l // 8
                col = (l % 8) * EMB
                copies.append(pltpu.async_copy(
                    tables[f].at[vec[l]],
                    rows_v.at[f, row, pl.ds(col, EMB)],
                    sem))
        for l in range(16):
            b = base + g * 16 + l
            row = g * 2 + l // 8
            col = (l % 8) * EMB
            copies.append(pltpu.async_copy(
                dn16.at[b],
                dnp_v.at[row, pl.ds(col, EMB)],
                sem))
        for c in copies:
            c.wait()
        return carry

    lax.fori_loop(0, GROUPS, step, 0, unroll=False)
    for f in range(4):
        pltpu.sync_copy(rows_v.at[f], emb_out.at[f, pl.ds(orow, PRW)])
    pltpu.sync_copy(dnp_v, dnp_out.at[pl.ds(orow, PRW)])


def _sc_gather(e_base, e_level, e_sex, e_tag,
               i_base, i_level, i_sex, i_tag, dn16):
    mesh = plsc.VectorSubcoreMesh(core_axis_name="c", subcore_axis_name="s")
    return pl.kernel(
        _sc_gather_body,
        mesh=mesh,
        out_type=[
            jax.ShapeDtypeStruct((4, PR, 128), jnp.float32),
            jax.ShapeDtypeStruct((PR, 128), jnp.float32),
        ],
        scratch_types=[
            pltpu.VMEM((4, BPW), jnp.int32),
            pltpu.VMEM((4, PRW, 128), jnp.float32),
            pltpu.VMEM((PRW, 128), jnp.float32),
            pltpu.SemaphoreType.DMA,
        ],
        compiler_params=pltpu.CompilerParams(use_tc_tiling_on_sc=False),
    )(e_base, e_level, e_sex, e_tag, i_base, i_level, i_sex, i_tag, dn16)


def _tc_dense_body(emb_ref, dnp_ref, tgt_ref,
                   w1f_ref, w1d_ref, b1_ref, w2_ref, b2_ref, w3_ref,
                   sel_ref, pred_ref, loss_ref):
    e0 = emb_ref[0]
    e1 = emb_ref[1]
    e2 = emb_ref[2]
    e3 = emb_ref[3]
    s = e0 + e1 + e2 + e3
    sq = e0 * e0 + e1 * e1 + e2 * e2 + e3 * e3
    cross = 0.5 * jnp.dot(s * s - sq, sel_ref[...],
                          preferred_element_type=jnp.float32)
    h = (jnp.dot(e0, w1f_ref[0], preferred_element_type=jnp.float32)
         + jnp.dot(e1, w1f_ref[1], preferred_element_type=jnp.float32)
         + jnp.dot(e2, w1f_ref[2], preferred_element_type=jnp.float32)
         + jnp.dot(e3, w1f_ref[3], preferred_element_type=jnp.float32)
         + jnp.dot(dnp_ref[...], w1d_ref[...],
                   preferred_element_type=jnp.float32)
         + b1_ref[...][None, :])
    h = jnp.maximum(h, 0.0)
    h = jnp.dot(h, w2_ref[...], preferred_element_type=jnp.float32) \
        + b2_ref[...][None, :]
    h = jnp.maximum(h, 0.0)
    logit = jnp.dot(h, w3_ref[...],
                    preferred_element_type=jnp.float32) + cross
    pred = 1.0 / (1.0 + jnp.exp(-logit))
    pred_ref[...] = pred
    p = jnp.clip(pred, 1e-7, 1.0 - 1e-7)
    t = tgt_ref[...]
    loss_ref[...] = (-jnp.sum(
        t * jnp.log(p) + (1.0 - t) * jnp.log(1.0 - p))
        * (1.0 / B)).reshape(1, 1)


def _tc_dense(emb, dnp, tgt2, w1f, w1d, b1bd, w2bd, b2bd, w3bd, sel):
    return pl.pallas_call(
        _tc_dense_body,
        out_shape=[
            jax.ShapeDtypeStruct((PR, 8), jnp.float32),
            jax.ShapeDtypeStruct((1, 1), jnp.float32),
        ],
    )(emb, dnp, tgt2, w1f, w1d, b1bd, w2bd, b2bd, w3bd, sel)


def kernel(base_cd, level, sex, tag, dense_features, target,
           E_base, E_level, E_sex, E_tag, W1, b1, W2, b2, W3):
    dn16 = jnp.pad(dense_features, ((0, 0), (0, EMB - N_DENSE)))
    emb, dnp = _sc_gather(
        E_base, E_level, E_sex, E_tag,
        base_cd.astype(jnp.int32), level.astype(jnp.int32),
        sex.astype(jnp.int32), tag.astype(jnp.int32), dn16)
    eye8 = jnp.eye(8, dtype=jnp.float32)
    w1f = jnp.stack([
        jnp.kron(eye8, W1[0:16]),
        jnp.kron(eye8, W1[16:32]),
        jnp.kron(eye8, W1[32:48]),
        jnp.kron(eye8, W1[48:64]),
    ])
    w1d = jnp.kron(eye8, jnp.pad(W1[64:77], ((0, 3), (0, 0))))
    b1bd = jnp.tile(b1, 8)
    w2bd = jnp.kron(eye8, W2)
    b2bd = jnp.tile(b2, 8)
    w3bd = jnp.kron(eye8, W3)
    sel = jnp.kron(eye8, jnp.ones((EMB, 1), jnp.float32))
    tgt2 = target.reshape(PR, 8)
    pred_p, loss = _tc_dense(emb, dnp, tgt2, w1f, w1d, b1bd, w2bd, b2bd,
                             w3bd, sel)
    return (pred_p.reshape(B), loss[0, 0])
